# baseline (speedup 1.0000x reference)
import functools

import jax
import jax.numpy as jnp
from jax import lax
from jax.experimental import pallas as pl
from jax.experimental.pallas import tpu as pltpu
from jax.experimental.pallas import tpu_sc as plsc

EPS = 1e-5
KNN = 32
B, N = 4, 4096
NT = B * N
NC, NS = 2, 16
NW = NC * NS
RPW = NT // NW
CAP = N + 16
BIGF = 3e38
BIGI = 1 << 30


def _dist_body(ptn_ref, pm_ref, sel_ref, dist_ref, d8_ref, t_ref):
    ptn = ptn_ref[0]
    pm = pm_ref[0]
    inner = jnp.dot(ptn.astype(jnp.bfloat16), pm.astype(jnp.bfloat16),
                    preferred_element_type=jnp.float32)
    sq_n = (ptn[:, 0:1] * ptn[:, 0:1] + ptn[:, 1:2] * ptn[:, 1:2]
            + ptn[:, 2:3] * ptn[:, 2:3])
    sq_m = (pm[0:1, :] * pm[0:1, :] + pm[1:2, :] * pm[1:2, :]
            + pm[2:3, :] * pm[2:3, :])
    dist = sq_n - 2.0 * inner + sq_m
    dist_ref[0] = dist
    t3 = dist
    for s in (1, 2, 4):
        t3 = jnp.minimum(t3, pltpu.roll(t3, N - s, 1))
    d8 = jnp.dot(t3.astype(jnp.bfloat16), sel_ref[...],
                 preferred_element_type=jnp.float32)
    d8_ref[...] = d8
    t = None
    for g in range(32):
        gm = jnp.min(d8[:, g * 16:(g + 1) * 16], axis=1, keepdims=True)
        t = gm if t is None else jnp.maximum(t, gm)
    t_ref[...] = t + jnp.abs(t) * 0.0078125 + 1e-4


def _pairwise_dist(points):
    R = 512
    pts_pad = jnp.concatenate(
        [points, jnp.zeros((B, 8 - 3, N), points.dtype)], axis=1)
    ptsT = jnp.transpose(pts_pad, (0, 2, 1))
    sel = (lax.broadcasted_iota(jnp.int32, (N, N // 8), 0)
           == 8 * lax.broadcasted_iota(jnp.int32, (N, N // 8), 1)
           ).astype(jnp.bfloat16)
    return pl.pallas_call(
        _dist_body,
        grid=(B, N // R),
        in_specs=[
            pl.BlockSpec((1, R, 8), lambda b, i: (b, i, 0)),
            pl.BlockSpec((1, 8, N), lambda b, i: (b, 0, 0)),
            pl.BlockSpec((N, N // 8), lambda b, i: (0, 0)),
        ],
        out_specs=[
            pl.BlockSpec((1, R, N), lambda b, i: (b, i, 0)),
            pl.BlockSpec((R, N // 8), lambda b, i: (b * (N // R) + i, 0)),
            pl.BlockSpec((R, 1), lambda b, i: (b * (N // R) + i, 0)),
        ],
        out_shape=[
            jax.ShapeDtypeStruct((B, N, N), jnp.float32),
            jax.ShapeDtypeStruct((NT, N // 8), jnp.float32),
            jax.ShapeDtypeStruct((NT, 1), jnp.float32),
        ],
    )(ptsT, pts_pad, sel)


def _topk_body(dist_hbm, d8_hbm, t_hbm, out_hbm, rowbuf, d8buf, tbuf, glist,
               cval, cidx, minbuf, obuf):
    wid = lax.axis_index("s") * NC + lax.axis_index("c")
    base_row = pl.multiple_of(wid * RPW, RPW)
    pltpu.sync_copy(t_hbm.at[pl.ds(base_row, RPW)], tbuf)
    iota = lax.iota(jnp.int32, 16)

    def one_row(rb, r):
        tsp = plsc.load_gather(tbuf, [jnp.full((16,), r, jnp.int32)])

        def gselbody(i, gptr):
            for u in range(4):
                ii = i * 4 + u
                v = d8buf[rb, pl.ds(ii * 16, 16)]
                m = v <= tsp
                plsc.store_compressed(glist.at[pl.ds(gptr, 16)],
                                      iota + ii * 16, mask=m)
                cnt = plsc.all_reduce_population_count(m)
                gptr = gptr + cnt[0]
            return gptr

        ng = lax.fori_loop(0, N // 8 // 64, gselbody, jnp.int32(0))
        plsc.store_scatter(glist, [ng + iota], jnp.zeros((16,), jnp.int32))

        lane_g = (iota >= 8).astype(jnp.int32)
        lane_o = iota & 7

        rbsp = jnp.full((16,), rb, jnp.int32)

        def selbody(j, ptr):
            for u in range(2):
                gpos = (j * 2 + u) * 2 + lane_g
                gid = plsc.load_gather(glist, [gpos])
                idx = gid * 8 + lane_o
                v = plsc.load_gather(rowbuf, [rbsp, idx])
                m = (v <= tsp) & (gpos < ng)
                plsc.store_compressed(cidx.at[pl.ds(ptr, 16)], idx, mask=m)
                plsc.store_compressed(cval.at[pl.ds(ptr, 16)], v, mask=m)
                cnt = plsc.all_reduce_population_count(m)
                ptr = ptr + cnt[0]
            return ptr

        ptr = lax.fori_loop(0, (ng + 3) // 4, selbody, jnp.int32(0))
        plsc.store_scatter(cval, [ptr + iota],
                           jnp.full((16,), BIGF, jnp.float32))
        nchunk = (ptr + 15) // 16

        dnums = lax.GatherDimensionNumbers(
            offset_dims=(), collapsed_slice_dims=(0,), start_index_map=(0,))

        def _shuf(v, idx16):
            return lax.gather(
                v, idx16[:, None], dnums, slice_sizes=(1,),
                mode=lax.GatherScatterMode.PROMISE_IN_BOUNDS)

        def _splat_min(v):
            for s in (1, 2, 4, 8):
                v = jnp.minimum(v, _shuf(v, iota ^ s))
            return v

        def minbody(ci, _):
            v = cval[pl.ds(ci * 16, 16)]
            mn = _splat_min(v)
            plsc.store_scatter(minbuf, [jnp.full((16,), ci, jnp.int32)],
                               mn, mask=(iota == 0))
            return 0

        lax.fori_loop(0, nchunk, minbody, 0)
        plsc.store_scatter(minbuf, [nchunk + iota],
                           jnp.full((16,), BIGF, jnp.float32))

        def tbody(t, _):
            def mscan(ci, mc):
                m, am = mc
                v = minbuf[pl.ds(ci * 16, 16)]
                am = jnp.where(v < m, jnp.full((16,), ci, jnp.int32), am)
                return (jnp.minimum(m, v), am)

            m, am = lax.fori_loop(
                0, (nchunk + 15) // 16, mscan,
                (jnp.full((16,), BIGF, jnp.float32),
                 jnp.zeros((16,), jnp.int32)))
            rminv = _splat_min(m)
            posc = jnp.where(m == rminv, am * 16 + iota,
                             jnp.full((16,), BIGI, jnp.int32))
            cstar = _splat_min(posc)[0]
            v = cval[pl.ds(cstar * 16, 16)]
            posl = jnp.where(v == rminv, iota,
                             jnp.full((16,), BIGI, jnp.int32))
            lv = _splat_min(posl)
            psp = cstar * 16 + lv
            pidx = plsc.load_gather(cidx, [psp])
            plsc.store_scatter(obuf, [jnp.full((16,), rb * KNN + t, jnp.int32)],
                               pidx, mask=(iota == 0))
            nv = jnp.where(iota == lv, jnp.full((16,), BIGF, jnp.float32), v)
            cval[pl.ds(cstar * 16, 16)] = nv
            plsc.store_scatter(minbuf, [jnp.full((16,), cstar, jnp.int32)],
                               _splat_min(nv), mask=(iota == 0))
            return 0

        lax.fori_loop(0, KNN, tbody, 0)

    def group_body(gidx8, _):
        row8 = pl.multiple_of(base_row + gidx8 * 8, 8)
        pltpu.sync_copy(dist_hbm.at[pl.ds(row8, 8)], rowbuf)
        pltpu.sync_copy(d8_hbm.at[pl.ds(row8, 8)], d8buf)
        for rb in range(8):
            one_row(rb, gidx8 * 8 + rb)
        gbase = (row8 // N) * N
        for i in range(8 * KNN // 16):
            obuf[pl.ds(i * 16, 16)] = obuf[pl.ds(i * 16, 16)] + gbase
        pltpu.sync_copy(
            obuf, out_hbm.at[pl.ds(pl.multiple_of(row8 * KNN, 8), 8 * KNN)])
        return 0

    lax.fori_loop(0, RPW // 8, group_body, 0)


def _sc_topk(dist2d, d8, t1d):
    mesh = plsc.VectorSubcoreMesh(core_axis_name="c", subcore_axis_name="s")
    return pl.kernel(
        _topk_body,
        out_type=jax.ShapeDtypeStruct((NT * KNN,), jnp.int32),
        mesh=mesh,
        scratch_types=[
            pltpu.VMEM((8, N), jnp.float32),
            pltpu.VMEM((8, N // 8), jnp.float32),
            pltpu.VMEM((RPW,), jnp.float32),
            pltpu.VMEM((N // 8 + 32,), jnp.int32),
            pltpu.VMEM((CAP,), jnp.float32),
            pltpu.VMEM((CAP,), jnp.int32),
            pltpu.VMEM((CAP // 16 + 32,), jnp.float32),
            pltpu.VMEM((8 * KNN,), jnp.int32),
        ],
        compiler_params=pltpu.CompilerParams(needs_layout_passes=False),
    )(dist2d, d8, t1d)


def _gather_body(x_hbm, gidx_hbm, g_hbm, idxbuf, rows, sem, c, chunk):
    wid = lax.axis_index("s") * NC + lax.axis_index("c")
    total = NT * KNN // NW
    wbase = pl.multiple_of(wid * total, total)
    nstream = chunk // 128
    pltpu.sync_copy(
        gidx_hbm.at[pl.ds(pl.multiple_of(wbase // 128, 8), total // 128)],
        idxbuf)

    def chunk_body(j, _):
        cps = [
            pltpu.async_copy(
                x_hbm.at[idxbuf.at[j * nstream + s]],
                rows.at[pl.ds(s * 128, 128)], sem)
            for s in range(nstream)
        ]
        for cp in cps:
            cp.wait()
        pltpu.sync_copy(
            rows,
            g_hbm.at[pl.ds(pl.multiple_of(wbase + j * chunk, chunk), chunk)])
        return 0

    lax.fori_loop(0, total // chunk, chunk_body, 0)


def _sc_gather(x_rows, gidx2d, c, chunk):
    mesh = plsc.VectorSubcoreMesh(core_axis_name="c", subcore_axis_name="s")
    body = functools.partial(_gather_body, c=c, chunk=chunk)
    return pl.kernel(
        body,
        out_type=jax.ShapeDtypeStruct((NT * KNN, c), jnp.float32),
        mesh=mesh,
        scratch_types=[
            pltpu.VMEM((NT * KNN // NW // 128, 128), jnp.int32),
            pltpu.VMEM((chunk, c), jnp.float32),
            pltpu.SemaphoreType.DMA,
        ],
        compiler_params=pltpu.CompilerParams(
            needs_layout_passes=False, use_tc_tiling_on_sc=False),
    )(x_rows, gidx2d)


def _edgeconv_body(g_ref, x_ref, w_ref, ymax_ref, stats_ref, acc, *, R, c, o):
    pid = pl.program_id(0)
    gb = g_ref[...]
    xb = x_ref[...]
    w = w_ref[...]
    wa = w[:, :c]
    wb = w[:, c:]
    d3 = gb.reshape(R, KNN, c) - xb[:, None, :]
    d = d3.astype(jnp.bfloat16).reshape(R * KNN, c)
    yd = lax.dot_general(d, wa, (((1,), (1,)), ((), ())),
                         preferred_element_type=jnp.float32)
    cen = lax.dot_general(xb.astype(jnp.bfloat16), wb,
                          (((1,), (1,)), ((), ())),
                          preferred_element_type=jnp.float32)
    y3 = yd.reshape(R, KNN, o) + cen[:, None, :]
    ymax_ref[...] = jnp.max(y3, axis=1)
    y2 = y3.reshape(R * KNN, o)
    s = jnp.sum(y2, axis=0, keepdims=True)
    q = jnp.sum(y2 * y2, axis=0, keepdims=True)

    @pl.when(pid == 0)
    def _():
        acc[...] = jnp.zeros_like(acc)

    acc[0:1, :] += s
    acc[1:2, :] += q

    @pl.when(pid == pl.num_programs(0) - 1)
    def _():
        stats_ref[...] = acc[0:2, :]


def _edgeconv(g, x_rows, w_bf, c, o):
    R = 256
    body = functools.partial(_edgeconv_body, R=R, c=c, o=o)
    return pl.pallas_call(
        body,
        grid=(NT // R,),
        in_specs=[
            pl.BlockSpec((R * KNN, c), lambda i: (i, 0)),
            pl.BlockSpec((R, c), lambda i: (i, 0)),
            pl.BlockSpec((o, 2 * c), lambda i: (0, 0)),
        ],
        out_specs=[
            pl.BlockSpec((R, o), lambda i: (i, 0)),
            pl.BlockSpec((2, o), lambda i: (0, 0)),
        ],
        out_shape=[
            jax.ShapeDtypeStruct((NT, o), jnp.float32),
            jax.ShapeDtypeStruct((2, o), jnp.float32),
        ],
        scratch_shapes=[pltpu.VMEM((8, o), jnp.float32)],
    )(g, x_rows, w_bf)


def _bn_body(y_ref, stats_ref, g_ref, b_ref, o_ref, *, count):
    s = stats_ref[0:1, :]
    q = stats_ref[1:2, :]
    mean = s * (1.0 / count)
    var = q * (1.0 / count) - mean * mean
    xn = g_ref[...] * (y_ref[...] - mean) / jnp.sqrt(var + EPS) + b_ref[...]
    o_ref[...] = jnp.where(xn >= 0, xn, 0.2 * xn)


def _bn_apply(y, stats, g, b, count):
    R = 2048
    o = y.shape[1]
    body = functools.partial(_bn_body, count=count)
    return pl.pallas_call(
        body,
        grid=(NT // R,),
        in_specs=[
            pl.BlockSpec((R, o), lambda i: (i, 0)),
            pl.BlockSpec((2, o), lambda i: (0, 0)),
            pl.BlockSpec((1, o), lambda i: (0, 0)),
            pl.BlockSpec((1, o), lambda i: (0, 0)),
        ],
        out_specs=pl.BlockSpec((R, o), lambda i: (i, 0)),
        out_shape=jax.ShapeDtypeStruct((NT, o), jnp.float32),
    )(y, stats, g.reshape(1, o), b.reshape(1, o))


def _outconv_body(x1_ref, x2_ref, x3_ref, w_ref, y_ref, stats_ref, acc):
    pid = pl.program_id(0)
    feat = jnp.concatenate(
        [x1_ref[...], x2_ref[...], x3_ref[...]], axis=1)
    y = lax.dot_general(feat.astype(jnp.bfloat16), w_ref[...],
                        (((1,), (1,)), ((), ())),
                        preferred_element_type=jnp.float32)
    y_ref[...] = y
    s = jnp.sum(y, axis=0, keepdims=True)
    q = jnp.sum(y * y, axis=0, keepdims=True)

    @pl.when(pid == 0)
    def _():
        acc[...] = jnp.zeros_like(acc)

    acc[0:1, :] += s
    acc[1:2, :] += q

    @pl.when(pid == pl.num_programs(0) - 1)
    def _():
        stats_ref[...] = acc[0:2, :]


def _outconv(x1, x2, x3, w_bf):
    R = 1024
    return pl.pallas_call(
        _outconv_body,
        grid=(NT // R,),
        in_specs=[
            pl.BlockSpec((R, 64), lambda i: (i, 0)),
            pl.BlockSpec((R, 64), lambda i: (i, 0)),
            pl.BlockSpec((R, 128), lambda i: (i, 0)),
            pl.BlockSpec((256, 256), lambda i: (0, 0)),
        ],
        out_specs=[
            pl.BlockSpec((R, 256), lambda i: (i, 0)),
            pl.BlockSpec((2, 256), lambda i: (0, 0)),
        ],
        out_shape=[
            jax.ShapeDtypeStruct((NT, 256), jnp.float32),
            jax.ShapeDtypeStruct((2, 256), jnp.float32),
        ],
        scratch_shapes=[pltpu.VMEM((8, 256), jnp.float32)],
    )(x1, x2, x3, w_bf)


def kernel(feats, points, W0, W1, W2, W_out, g0, b0, g1, b1, g2, b2, g_out, b_out):
    dist, d8, t = _pairwise_dist(points)
    gidx = _sc_topk(dist.reshape(NT, N), d8, t.reshape(NT))

    gidx_pm = gidx.reshape(NT * KNN // 128, 128)

    x = jnp.transpose(feats, (0, 2, 1)).reshape(NT, 64)
    outs = []
    for W, g, b in ((W0, g0, b0), (W1, g1, b1), (W2, g2, b2)):
        c = x.shape[1]
        o = W.shape[0]
        gpm = _sc_gather(x, gidx_pm, c, 512 if c == 64 else 256)
        ymax, stats = _edgeconv(gpm, x, W.astype(jnp.bfloat16), c, o)
        x = _bn_apply(ymax, stats, g, b, NT * KNN)
        outs.append(x)

    y, stats = _outconv(outs[0], outs[1], outs[2], W_out.astype(jnp.bfloat16))
    out = _bn_apply(y, stats, g_out, b_out, NT)
    return jnp.transpose(out.reshape(B, N, 256), (0, 2, 1))


# speedup vs baseline: 1.0019x; 1.0019x over previous
import functools

import jax
import jax.numpy as jnp
from jax import lax
from jax.experimental import pallas as pl
from jax.experimental.pallas import tpu as pltpu
from jax.experimental.pallas import tpu_sc as plsc

EPS = 1e-5
KNN = 32
B, N = 4, 4096
NT = B * N
NC, NS = 2, 16
NW = NC * NS
RPW = NT // NW
CAP = N + 16
BIGF = 3e38
BIGI = 1 << 30


def _dist_body(ptn_ref, pm_ref, sel_ref, dist_ref, d8_ref, t_ref):
    ptn = ptn_ref[0]
    pm = pm_ref[0]
    inner = jnp.dot(ptn.astype(jnp.bfloat16), pm.astype(jnp.bfloat16),
                    preferred_element_type=jnp.float32)
    sq_n = (ptn[:, 0:1] * ptn[:, 0:1] + ptn[:, 1:2] * ptn[:, 1:2]
            + ptn[:, 2:3] * ptn[:, 2:3])
    sq_m = (pm[0:1, :] * pm[0:1, :] + pm[1:2, :] * pm[1:2, :]
            + pm[2:3, :] * pm[2:3, :])
    dist = sq_n - 2.0 * inner + sq_m
    dist_ref[0] = dist
    t3 = dist
    for s in (1, 2, 4):
        t3 = jnp.minimum(t3, pltpu.roll(t3, N - s, 1))
    d8 = jnp.dot(t3.astype(jnp.bfloat16), sel_ref[...],
                 preferred_element_type=jnp.float32)
    d8_ref[...] = d8
    t = None
    for g in range(32):
        gm = jnp.min(d8[:, g * 16:(g + 1) * 16], axis=1, keepdims=True)
        t = gm if t is None else jnp.maximum(t, gm)
    t_ref[...] = t + jnp.abs(t) * 0.0078125 + 1e-4


def _pairwise_dist(points):
    R = 512
    pts_pad = jnp.concatenate(
        [points, jnp.zeros((B, 8 - 3, N), points.dtype)], axis=1)
    ptsT = jnp.transpose(pts_pad, (0, 2, 1))
    sel = (lax.broadcasted_iota(jnp.int32, (N, N // 8), 0)
           == 8 * lax.broadcasted_iota(jnp.int32, (N, N // 8), 1)
           ).astype(jnp.bfloat16)
    return pl.pallas_call(
        _dist_body,
        grid=(B, N // R),
        in_specs=[
            pl.BlockSpec((1, R, 8), lambda b, i: (b, i, 0)),
            pl.BlockSpec((1, 8, N), lambda b, i: (b, 0, 0)),
            pl.BlockSpec((N, N // 8), lambda b, i: (0, 0)),
        ],
        out_specs=[
            pl.BlockSpec((1, R, N), lambda b, i: (b, i, 0)),
            pl.BlockSpec((R, N // 8), lambda b, i: (b * (N // R) + i, 0)),
            pl.BlockSpec((R, 1), lambda b, i: (b * (N // R) + i, 0)),
        ],
        out_shape=[
            jax.ShapeDtypeStruct((B, N, N), jnp.float32),
            jax.ShapeDtypeStruct((NT, N // 8), jnp.float32),
            jax.ShapeDtypeStruct((NT, 1), jnp.float32),
        ],
    )(ptsT, pts_pad, sel)


def _topk_body(dist_hbm, d8_hbm, t_hbm, out_hbm, rowbuf, d8buf, tbuf, glist,
               cval, cidx, minbuf, obuf):
    wid = lax.axis_index("s") * NC + lax.axis_index("c")
    base_row = pl.multiple_of(wid * RPW, RPW)
    pltpu.sync_copy(t_hbm.at[pl.ds(base_row, RPW)], tbuf)
    iota = lax.iota(jnp.int32, 16)

    def one_row(rb, r):
        tsp = plsc.load_gather(tbuf, [jnp.full((16,), r, jnp.int32)])

        def gselbody(i, gptr):
            for u in range(4):
                ii = i * 4 + u
                v = d8buf[rb, pl.ds(ii * 16, 16)]
                m = v <= tsp
                plsc.store_compressed(glist.at[pl.ds(gptr, 16)],
                                      iota + ii * 16, mask=m)
                cnt = plsc.all_reduce_population_count(m)
                gptr = gptr + cnt[0]
            return gptr

        ng = lax.fori_loop(0, N // 8 // 64, gselbody, jnp.int32(0))
        plsc.store_scatter(glist, [ng + iota], jnp.zeros((16,), jnp.int32))

        lane_g = (iota >= 8).astype(jnp.int32)
        lane_o = iota & 7

        def selbody(j, ptr):
            gpos = j * 2 + lane_g
            gid = plsc.load_gather(glist, [gpos])
            idx = gid * 8 + lane_o
            v = plsc.load_gather(rowbuf,
                                 [jnp.full((16,), rb, jnp.int32), idx])
            m = (v <= tsp) & (gpos < ng)
            plsc.store_compressed(cidx.at[pl.ds(ptr, 16)], idx, mask=m)
            plsc.store_compressed(cval.at[pl.ds(ptr, 16)], v, mask=m)
            cnt = plsc.all_reduce_population_count(m)
            return ptr + cnt[0]

        ptr = lax.fori_loop(0, (ng + 1) // 2, selbody, jnp.int32(0))
        plsc.store_scatter(cval, [ptr + iota],
                           jnp.full((16,), BIGF, jnp.float32))
        nchunk = (ptr + 15) // 16

        dnums = lax.GatherDimensionNumbers(
            offset_dims=(), collapsed_slice_dims=(0,), start_index_map=(0,))

        def _shuf(v, idx16):
            return lax.gather(
                v, idx16[:, None], dnums, slice_sizes=(1,),
                mode=lax.GatherScatterMode.PROMISE_IN_BOUNDS)

        def _splat_min(v):
            for s in (1, 2, 4, 8):
                v = jnp.minimum(v, _shuf(v, iota ^ s))
            return v

        def minbody(ci, _):
            v = cval[pl.ds(ci * 16, 16)]
            mn = _splat_min(v)
            plsc.store_scatter(minbuf, [jnp.full((16,), ci, jnp.int32)],
                               mn, mask=(iota == 0))
            return 0

        lax.fori_loop(0, nchunk, minbody, 0)
        plsc.store_scatter(minbuf, [nchunk + iota],
                           jnp.full((16,), BIGF, jnp.float32))

        def tbody(t, _):
            def mscan(ci, mc):
                m, am = mc
                v = minbuf[pl.ds(ci * 16, 16)]
                am = jnp.where(v < m, jnp.full((16,), ci, jnp.int32), am)
                return (jnp.minimum(m, v), am)

            m, am = lax.fori_loop(
                0, (nchunk + 15) // 16, mscan,
                (jnp.full((16,), BIGF, jnp.float32),
                 jnp.zeros((16,), jnp.int32)))
            rminv = _splat_min(m)
            posc = jnp.where(m == rminv, am * 16 + iota,
                             jnp.full((16,), BIGI, jnp.int32))
            cstar = _splat_min(posc)[0]
            v = cval[pl.ds(cstar * 16, 16)]
            posl = jnp.where(v == rminv, iota,
                             jnp.full((16,), BIGI, jnp.int32))
            lv = _splat_min(posl)
            psp = cstar * 16 + lv
            pidx = plsc.load_gather(cidx, [psp])
            plsc.store_scatter(obuf, [jnp.full((16,), rb * KNN + t, jnp.int32)],
                               pidx, mask=(iota == 0))
            nv = jnp.where(iota == lv, jnp.full((16,), BIGF, jnp.float32), v)
            cval[pl.ds(cstar * 16, 16)] = nv
            plsc.store_scatter(minbuf, [jnp.full((16,), cstar, jnp.int32)],
                               _splat_min(nv), mask=(iota == 0))
            return 0

        lax.fori_loop(0, KNN, tbody, 0)

    def group_body(gidx8, _):
        row8 = pl.multiple_of(base_row + gidx8 * 8, 8)
        pltpu.sync_copy(dist_hbm.at[pl.ds(row8, 8)], rowbuf)
        pltpu.sync_copy(d8_hbm.at[pl.ds(row8, 8)], d8buf)
        for rb in range(8):
            one_row(rb, gidx8 * 8 + rb)
        gbase = (row8 // N) * N
        for i in range(8 * KNN // 16):
            obuf[pl.ds(i * 16, 16)] = obuf[pl.ds(i * 16, 16)] + gbase
        pltpu.sync_copy(
            obuf, out_hbm.at[pl.ds(pl.multiple_of(row8 * KNN, 8), 8 * KNN)])
        return 0

    lax.fori_loop(0, RPW // 8, group_body, 0)


def _sc_topk(dist2d, d8, t1d):
    mesh = plsc.VectorSubcoreMesh(core_axis_name="c", subcore_axis_name="s")
    return pl.kernel(
        _topk_body,
        out_type=jax.ShapeDtypeStruct((NT * KNN,), jnp.int32),
        mesh=mesh,
        scratch_types=[
            pltpu.VMEM((8, N), jnp.float32),
            pltpu.VMEM((8, N // 8), jnp.float32),
            pltpu.VMEM((RPW,), jnp.float32),
            pltpu.VMEM((N // 8 + 32,), jnp.int32),
            pltpu.VMEM((CAP,), jnp.float32),
            pltpu.VMEM((CAP,), jnp.int32),
            pltpu.VMEM((CAP // 16 + 32,), jnp.float32),
            pltpu.VMEM((8 * KNN,), jnp.int32),
        ],
        compiler_params=pltpu.CompilerParams(needs_layout_passes=False),
    )(dist2d, d8, t1d)


def _gather_body(x_hbm, gidx_hbm, g_hbm, idxbuf, rows, sem, c, chunk):
    wid = lax.axis_index("s") * NC + lax.axis_index("c")
    total = NT * KNN // NW
    wbase = pl.multiple_of(wid * total, total)
    nstream = chunk // 128
    pltpu.sync_copy(
        gidx_hbm.at[pl.ds(pl.multiple_of(wbase // 128, 8), total // 128)],
        idxbuf)

    def chunk_body(j, _):
        cps = [
            pltpu.async_copy(
                x_hbm.at[idxbuf.at[j * nstream + s]],
                rows.at[pl.ds(s * 128, 128)], sem)
            for s in range(nstream)
        ]
        for cp in cps:
            cp.wait()
        pltpu.sync_copy(
            rows,
            g_hbm.at[pl.ds(pl.multiple_of(wbase + j * chunk, chunk), chunk)])
        return 0

    lax.fori_loop(0, total // chunk, chunk_body, 0)


def _sc_gather(x_rows, gidx2d, c, chunk):
    mesh = plsc.VectorSubcoreMesh(core_axis_name="c", subcore_axis_name="s")
    body = functools.partial(_gather_body, c=c, chunk=chunk)
    return pl.kernel(
        body,
        out_type=jax.ShapeDtypeStruct((NT * KNN, c), jnp.float32),
        mesh=mesh,
        scratch_types=[
            pltpu.VMEM((NT * KNN // NW // 128, 128), jnp.int32),
            pltpu.VMEM((chunk, c), jnp.float32),
            pltpu.SemaphoreType.DMA,
        ],
        compiler_params=pltpu.CompilerParams(
            needs_layout_passes=False, use_tc_tiling_on_sc=False),
    )(x_rows, gidx2d)


def _edgeconv_body(g_ref, x_ref, w_ref, ymax_ref, stats_ref, acc, *, R, c, o):
    pid = pl.program_id(0)
    gb = g_ref[...]
    xb = x_ref[...]
    w = w_ref[...]
    wa = w[:, :c]
    wb = w[:, c:]
    d3 = gb.reshape(R, KNN, c) - xb[:, None, :]
    d = d3.astype(jnp.bfloat16).reshape(R * KNN, c)
    yd = lax.dot_general(d, wa, (((1,), (1,)), ((), ())),
                         preferred_element_type=jnp.float32)
    cen = lax.dot_general(xb.astype(jnp.bfloat16), wb,
                          (((1,), (1,)), ((), ())),
                          preferred_element_type=jnp.float32)
    y3 = yd.reshape(R, KNN, o) + cen[:, None, :]
    ymax_ref[...] = jnp.max(y3, axis=1)
    y2 = y3.reshape(R * KNN, o)
    s = jnp.sum(y2, axis=0, keepdims=True)
    q = jnp.sum(y2 * y2, axis=0, keepdims=True)

    @pl.when(pid == 0)
    def _():
        acc[...] = jnp.zeros_like(acc)

    acc[0:1, :] += s
    acc[1:2, :] += q

    @pl.when(pid == pl.num_programs(0) - 1)
    def _():
        stats_ref[...] = acc[0:2, :]


def _edgeconv(g, x_rows, w_bf, c, o):
    R = 256
    body = functools.partial(_edgeconv_body, R=R, c=c, o=o)
    return pl.pallas_call(
        body,
        grid=(NT // R,),
        in_specs=[
            pl.BlockSpec((R * KNN, c), lambda i: (i, 0)),
            pl.BlockSpec((R, c), lambda i: (i, 0)),
            pl.BlockSpec((o, 2 * c), lambda i: (0, 0)),
        ],
        out_specs=[
            pl.BlockSpec((R, o), lambda i: (i, 0)),
            pl.BlockSpec((2, o), lambda i: (0, 0)),
        ],
        out_shape=[
            jax.ShapeDtypeStruct((NT, o), jnp.float32),
            jax.ShapeDtypeStruct((2, o), jnp.float32),
        ],
        scratch_shapes=[pltpu.VMEM((8, o), jnp.float32)],
    )(g, x_rows, w_bf)


def _bn_body(y_ref, stats_ref, g_ref, b_ref, o_ref, *, count):
    s = stats_ref[0:1, :]
    q = stats_ref[1:2, :]
    mean = s * (1.0 / count)
    var = q * (1.0 / count) - mean * mean
    xn = g_ref[...] * (y_ref[...] - mean) / jnp.sqrt(var + EPS) + b_ref[...]
    o_ref[...] = jnp.where(xn >= 0, xn, 0.2 * xn)


def _bn_apply(y, stats, g, b, count):
    R = 2048
    o = y.shape[1]
    body = functools.partial(_bn_body, count=count)
    return pl.pallas_call(
        body,
        grid=(NT // R,),
        in_specs=[
            pl.BlockSpec((R, o), lambda i: (i, 0)),
            pl.BlockSpec((2, o), lambda i: (0, 0)),
            pl.BlockSpec((1, o), lambda i: (0, 0)),
            pl.BlockSpec((1, o), lambda i: (0, 0)),
        ],
        out_specs=pl.BlockSpec((R, o), lambda i: (i, 0)),
        out_shape=jax.ShapeDtypeStruct((NT, o), jnp.float32),
    )(y, stats, g.reshape(1, o), b.reshape(1, o))


def _outconv_body(x1_ref, x2_ref, x3_ref, w_ref, y_ref, stats_ref, acc):
    pid = pl.program_id(0)
    feat = jnp.concatenate(
        [x1_ref[...], x2_ref[...], x3_ref[...]], axis=1)
    y = lax.dot_general(feat.astype(jnp.bfloat16), w_ref[...],
                        (((1,), (1,)), ((), ())),
                        preferred_element_type=jnp.float32)
    y_ref[...] = y
    s = jnp.sum(y, axis=0, keepdims=True)
    q = jnp.sum(y * y, axis=0, keepdims=True)

    @pl.when(pid == 0)
    def _():
        acc[...] = jnp.zeros_like(acc)

    acc[0:1, :] += s
    acc[1:2, :] += q

    @pl.when(pid == pl.num_programs(0) - 1)
    def _():
        stats_ref[...] = acc[0:2, :]


def _outconv(x1, x2, x3, w_bf):
    R = 1024
    return pl.pallas_call(
        _outconv_body,
        grid=(NT // R,),
        in_specs=[
            pl.BlockSpec((R, 64), lambda i: (i, 0)),
            pl.BlockSpec((R, 64), lambda i: (i, 0)),
            pl.BlockSpec((R, 128), lambda i: (i, 0)),
            pl.BlockSpec((256, 256), lambda i: (0, 0)),
        ],
        out_specs=[
            pl.BlockSpec((R, 256), lambda i: (i, 0)),
            pl.BlockSpec((2, 256), lambda i: (0, 0)),
        ],
        out_shape=[
            jax.ShapeDtypeStruct((NT, 256), jnp.float32),
            jax.ShapeDtypeStruct((2, 256), jnp.float32),
        ],
        scratch_shapes=[pltpu.VMEM((8, 256), jnp.float32)],
    )(x1, x2, x3, w_bf)


def kernel(feats, points, W0, W1, W2, W_out, g0, b0, g1, b1, g2, b2, g_out, b_out):
    dist, d8, t = _pairwise_dist(points)
    gidx = _sc_topk(dist.reshape(NT, N), d8, t.reshape(NT))

    gidx_pm = gidx.reshape(NT * KNN // 128, 128)

    x = jnp.transpose(feats, (0, 2, 1)).reshape(NT, 64)
    outs = []
    for W, g, b in ((W0, g0, b0), (W1, g1, b1), (W2, g2, b2)):
        c = x.shape[1]
        o = W.shape[0]
        gpm = _sc_gather(x, gidx_pm, c, 512 if c == 64 else 256)
        ymax, stats = _edgeconv(gpm, x, W.astype(jnp.bfloat16), c, o)
        x = _bn_apply(ymax, stats, g, b, NT * KNN)
        outs.append(x)

    y, stats = _outconv(outs[0], outs[1], outs[2], W_out.astype(jnp.bfloat16))
    out = _bn_apply(y, stats, g_out, b_out, NT)
    return jnp.transpose(out.reshape(B, N, 256), (0, 2, 1))
